# baseline (speedup 1.0000x reference)
import functools

import jax
import jax.numpy as jnp
from jax import lax
from jax.experimental import pallas as pl
from jax.experimental.pallas import tpu as pltpu
from jax.experimental.pallas import tpu_sc as plsc

_INPUT_DIM = 39
_N_CAT = 26
_VOCAB = 100000
_EMB = 16
_BATCH = 16384
_N_CONT = _INPUT_DIM - _N_CAT

_NC = 2
_NS = 16
_NW = _NC * _NS

_B_PER_W = _BATCH // _NW
_CHUNK = 128
_N_CHUNKS = _B_PER_W // _CHUNK

_L = 16


def _make_embed(j0, nj, with_cont):
    ncont = _N_CONT if with_cont else 0
    n_rows = ncont + nj * _EMB

    @functools.partial(
        pl.kernel,
        mesh=plsc.VectorSubcoreMesh(core_axis_name="c", subcore_axis_name="s"),
        out_type=jax.ShapeDtypeStruct((n_rows, _BATCH), jnp.float32),
        scratch_types=[
            pltpu.VMEM((_CHUNK, _INPUT_DIM), jnp.int32),
            pltpu.VMEM((nj * _CHUNK,), jnp.int32),
            pltpu.VMEM((n_rows, _CHUNK), jnp.float32),
            pltpu.SemaphoreType.DMA,
        ],
        compiler_params=pltpu.CompilerParams(
            use_tc_tiling_on_sc=False, needs_layout_passes=False
        ),
    )
    def _embed(x_hbm, tabt_hbm, out_hbm, x_v, idx_v, out_v, sem):
        wid = lax.axis_index("s") * _NC + lax.axis_index("c")
        w0 = wid * _B_PER_W
        iota = lax.iota(jnp.int32, _L)

        def chunk_body(c, carry):
            b0 = w0 + c * _CHUNK
            pltpu.sync_copy(x_hbm.at[pl.ds(b0, _CHUNK)], x_v)

            def feat_body(j, carry2):
                for g in range(_CHUNK // _L):
                    rb = g * _L + iota
                    r = plsc.load_gather(
                        x_v, [rb, iota * 0 + (_N_CONT + j0 + j)])
                    idx_v[pl.ds(j * _CHUNK + g * _L, _L)] = r
                for e in range(_EMB):
                    pltpu.async_copy(
                        tabt_hbm.at[j, e].at[
                            idx_v.at[pl.ds(j * _CHUNK, _CHUNK)]],
                        out_v.at[ncont + j * _EMB + e],
                        sem,
                    )
                return carry2

            lax.fori_loop(0, nj, feat_body, 0)

            for col in range(ncont):
                for g in range(_CHUNK // _L):
                    rb = g * _L + iota
                    vals = plsc.load_gather(x_v, [rb, iota * 0 + col])
                    out_v[col, pl.ds(g * _L, _L)] = vals.astype(jnp.float32)

            def drain_body(k, carry2):
                pltpu.make_async_copy(
                    tabt_hbm.at[0, 0, pl.ds(0, _CHUNK)],
                    out_v.at[ncont],
                    sem,
                ).wait()
                return carry2

            lax.fori_loop(0, nj * _EMB, drain_body, 0)

            pltpu.sync_copy(out_v, out_hbm.at[:, pl.ds(b0, _CHUNK)])
            return carry

        lax.fori_loop(0, _N_CHUNKS, chunk_body, 0)

    return _embed


_NJ_A = _N_CAT // 2
_embed_a = _make_embed(0, _NJ_A, True)
_embed_b = _make_embed(_NJ_A, _N_CAT - _NJ_A, False)


def kernel(x, tables):
    tab_t = tables.transpose(0, 2, 1)
    out_a = _embed_a(x, tab_t[:_NJ_A])
    out_b = _embed_b(x, tab_t[_NJ_A:])
    return jnp.concatenate([out_a, out_b], axis=0).T
